# Initial kernel scaffold; baseline (speedup 1.0000x reference)
import jax
import jax.numpy as jnp
from jax.experimental import pallas as pl


def kernel(z, codebook):
    raise NotImplementedError("write your pallas kernel here")


# speedup vs baseline: 2.1422x; 2.1422x over previous
import functools

import jax
import jax.numpy as jnp
from jax.experimental import pallas as pl
from jax.experimental.pallas import tpu as pltpu


def _vq_kernel(z_ref, cb_ref, qst_ref, idx_ref, ssq_ref):
    zb = z_ref[0]
    cb = cb_ref[...]

    scores = jax.lax.dot_general(
        cb, zb, (((1,), (0,)), ((), ())),
        preferred_element_type=jnp.float32)
    z2 = jnp.sum(zb * zb, axis=0, keepdims=True)
    c2 = jnp.sum(cb * cb, axis=1, keepdims=True)
    d = z2 - 2.0 * scores + c2

    idx = jnp.argmin(d, axis=0)
    idx_ref[0, 0, :] = idx

    onehot = (jax.lax.broadcasted_iota(jnp.int32, d.shape, 0)
              == idx[None, :]).astype(jnp.float32)
    qb = jax.lax.dot_general(
        cb, onehot, (((0,), (0,)), ((), ())),
        preferred_element_type=jnp.float32)

    diff = qb - zb
    qst_ref[0] = zb + diff

    @pl.when(pl.program_id(0) == 0)
    def _():
        ssq_ref[...] = jnp.zeros((1, 1), jnp.float32)
    ssq_ref[...] += jnp.sum(diff * diff).reshape(1, 1)


@jax.jit
def kernel(z, codebook):
    B, C, H, W = z.shape
    K = codebook.shape[0]
    P = H * W
    z3 = z.reshape(B, C, P)

    qst, idx, ssq = pl.pallas_call(
        _vq_kernel,
        grid=(B,),
        in_specs=[
            pl.BlockSpec((1, C, P), lambda b: (b, 0, 0)),
            pl.BlockSpec((K, C), lambda b: (0, 0)),
        ],
        out_specs=[
            pl.BlockSpec((1, C, P), lambda b: (b, 0, 0)),
            pl.BlockSpec((1, 1, P), lambda b: (b, 0, 0)),
            pl.BlockSpec((1, 1), lambda b: (0, 0)),
        ],
        out_shape=[
            jax.ShapeDtypeStruct((B, C, P), jnp.float32),
            jax.ShapeDtypeStruct((B, 1, P), jnp.int32),
            jax.ShapeDtypeStruct((1, 1), jnp.float32),
        ],
    )(z3, codebook)

    quantized_st = qst.reshape(B, C, H, W)
    encoding_indices = idx.reshape(B, H, W)
    mse = ssq[0, 0] / (B * C * H * W)
    commitment_loss = 0.25 * mse
    codebook_loss = mse
    return (quantized_st, encoding_indices, commitment_loss, codebook_loss)
